# initial kernel scaffold (unmeasured)
import jax
import jax.numpy as jnp
from jax import lax
from jax.experimental import pallas as pl
from jax.experimental.pallas import tpu as pltpu

N_DEV = 16
NSLOTS = 8
B, SQ, DM = 2, 256, 768
H, DH = 8, 64
SKV = 512
BH = B * H
SCALE = 0.125


def kernel(x, Wq, Wo, K_ext, V_ext):
    def body(x_ref, wq_ref, wo_ref, k_ref, v_ref, out_ref,
             kvbuf, send_sems, recv_sems):
        my = lax.axis_index("i")
        left = lax.rem(my + N_DEV - 1, N_DEV)
        right = lax.rem(my + 1, N_DEV)

        barrier_sem = pltpu.get_barrier_semaphore()
        for nbr in (left, right):
            pl.semaphore_signal(
                barrier_sem, inc=1,
                device_id=(nbr,), device_id_type=pl.DeviceIdType.MESH,
            )
        pl.semaphore_wait(barrier_sem, 2)

        for b in range(B):
            for h in range(H):
                kvbuf[0, 0, b * H + h] = k_ref[b, :, h, :].astype(jnp.bfloat16)
                kvbuf[0, 1, b * H + h] = v_ref[b, :, h, :].astype(jnp.bfloat16)

        xf = x_ref[...].reshape(B * SQ, DM).astype(jnp.bfloat16)
        q2d = lax.dot(
            xf, wq_ref[...].astype(jnp.bfloat16),
            preferred_element_type=jnp.float32,
        ).astype(jnp.bfloat16)
        q16 = jnp.concatenate(
            [q2d[b * SQ:(b + 1) * SQ, h * DH:(h + 1) * DH][None]
             for b in range(B) for h in range(H)],
            axis=0,
        )

        m = jnp.full((BH, SQ, 1), -jnp.inf, jnp.float32)
        l = jnp.zeros((BH, SQ, 1), jnp.float32)
        acc = jnp.zeros((BH, SQ, DH), jnp.float32)

        def chunk_update(slot, m, l, acc):
            kc = kvbuf[slot, 0]
            vc = kvbuf[slot, 1]
            s = lax.dot_general(
                q16, kc, (((2,), (2,)), ((0,), (0,))),
                preferred_element_type=jnp.float32,
            ) * SCALE
            mj = jnp.max(s, axis=-1, keepdims=True)
            m_new = jnp.maximum(m, mj)
            alpha = jnp.exp(m - m_new)
            p = jnp.exp(s - m_new)
            l_new = l * alpha + jnp.sum(p, axis=-1, keepdims=True)
            pv = lax.dot_general(
                p.astype(jnp.bfloat16), vc, (((2,), (1,)), ((0,), (0,))),
                preferred_element_type=jnp.float32,
            )
            return m_new, l_new, acc * alpha + pv

        m, l, acc = chunk_update(0, m, l, acc)

        for hop in range(N_DEV - 1):
            s_slot = hop % NSLOTS
            r_slot = (hop + 1) % NSLOTS
            rdma = pltpu.make_async_remote_copy(
                src_ref=kvbuf.at[s_slot],
                dst_ref=kvbuf.at[r_slot],
                send_sem=send_sems.at[hop],
                recv_sem=recv_sems.at[hop],
                device_id=(right,),
                device_id_type=pl.DeviceIdType.MESH,
            )
            rdma.start()
            rdma.wait()
            m, l, acc = chunk_update(r_slot, m, l, acc)

        o = acc / l
        for b in range(B):
            ob = jnp.concatenate([o[b * H + h] for h in range(H)], axis=1)
            out_ref[b] = lax.dot(
                ob.astype(jnp.bfloat16), wo_ref[...].astype(jnp.bfloat16),
                preferred_element_type=jnp.float32,
            )

    return pl.pallas_call(
        body,
        out_shape=jax.ShapeDtypeStruct((B, SQ, DM), jnp.float32),
        in_specs=[pl.BlockSpec(memory_space=pltpu.VMEM)] * 5,
        out_specs=pl.BlockSpec(memory_space=pltpu.VMEM),
        scratch_shapes=[
            pltpu.VMEM((NSLOTS, 2, BH, SKV, DH), jnp.bfloat16),
            pltpu.SemaphoreType.DMA((N_DEV - 1,)),
            pltpu.SemaphoreType.DMA((N_DEV - 1,)),
        ],
        compiler_params=pltpu.CompilerParams(collective_id=0),
    )(x, Wq, Wo, K_ext, V_ext)


# baseline (device time: 471727 ns/iter reference)
import os

import jax
import jax.numpy as jnp
from jax import lax
from jax.experimental import pallas as pl
from jax.experimental.pallas import tpu as pltpu

_CACHE_DIR = os.path.join(os.path.dirname(os.path.abspath(__file__)), ".jaxcache")
os.makedirs(_CACHE_DIR, exist_ok=True)
jax.config.update("jax_compilation_cache_dir", _CACHE_DIR)
jax.config.update("jax_persistent_cache_min_compile_time_secs", 1.0)

N_DEV = 16
NSLOTS = 4
B, SQ, DM = 2, 256, 768
H, DH = 8, 64
SKV = 512
BH = B * H
SCALE = 0.125


def kernel(x, Wq, Wo, K_ext, V_ext):
    def body(x_ref, wq_ref, wo_ref, k_ref, v_ref, out_ref,
             kvbuf, send_sems, recv_sems):
        my = lax.axis_index("i")
        left = lax.rem(my + N_DEV - 1, N_DEV)
        right = lax.rem(my + 1, N_DEV)

        barrier_sem = pltpu.get_barrier_semaphore()
        for nbr in (left, right):
            pl.semaphore_signal(
                barrier_sem, inc=1,
                device_id=(nbr,), device_id_type=pl.DeviceIdType.MESH,
            )
        pl.semaphore_wait(barrier_sem, 2)

        for b in range(B):
            for h in range(H):
                kvbuf[0, 0, b * H + h] = k_ref[b, :, h, :].astype(jnp.bfloat16).T
                kvbuf[0, 1, b * H + h] = v_ref[b, :, h, :].astype(jnp.bfloat16).T

        xf = x_ref[...].reshape(B * SQ, DM).astype(jnp.bfloat16)
        q2d = lax.dot(
            xf, wq_ref[...].astype(jnp.bfloat16),
            preferred_element_type=jnp.float32,
        ).astype(jnp.bfloat16)
        q16 = jnp.concatenate(
            [q2d[b * SQ:(b + 1) * SQ, h * DH:(h + 1) * DH][None]
             for b in range(B) for h in range(H)],
            axis=0,
        )

        m = jnp.full((BH, SQ, 1), -jnp.inf, jnp.float32)
        l = jnp.zeros((BH, SQ, 1), jnp.float32)
        acc = jnp.zeros((BH, SQ, DH), jnp.float32)

        def chunk_update(slot, m, l, acc):
            kc = kvbuf[slot, 0]
            vc = kvbuf[slot, 1]
            s = lax.dot_general(
                q16, kc, (((2,), (1,)), ((0,), (0,))),
                preferred_element_type=jnp.float32,
            ) * SCALE
            mj = jnp.max(s, axis=-1, keepdims=True)
            m_new = jnp.maximum(m, mj)
            alpha = jnp.exp(m - m_new)
            p = jnp.exp(s - m_new)
            l_new = l * alpha + jnp.sum(p, axis=-1, keepdims=True)
            pv = lax.dot_general(
                p.astype(jnp.bfloat16), vc, (((2,), (2,)), ((0,), (0,))),
                preferred_element_type=jnp.float32,
            )
            return m_new, l_new, acc * alpha + pv

        m, l, acc = chunk_update(0, m, l, acc)

        def hop_body(hop, carry):
            m, l, acc = carry
            s_slot = lax.rem(hop, NSLOTS)
            r_slot = lax.rem(hop + 1, NSLOTS)
            rdma = pltpu.make_async_remote_copy(
                src_ref=kvbuf.at[s_slot],
                dst_ref=kvbuf.at[r_slot],
                send_sem=send_sems.at[hop],
                recv_sem=recv_sems.at[hop],
                device_id=(right,),
                device_id_type=pl.DeviceIdType.MESH,
            )
            rdma.start()
            rdma.wait()
            return chunk_update(r_slot, m, l, acc)

        m, l, acc = lax.fori_loop(0, N_DEV - 1, hop_body, (m, l, acc))

        o = acc / l
        for b in range(B):
            ob = jnp.concatenate([o[b * H + h] for h in range(H)], axis=1)
            out_ref[b] = lax.dot(
                ob.astype(jnp.bfloat16), wo_ref[...].astype(jnp.bfloat16),
                preferred_element_type=jnp.float32,
            )

    return pl.pallas_call(
        body,
        out_shape=jax.ShapeDtypeStruct((B, SQ, DM), jnp.float32),
        in_specs=[pl.BlockSpec(memory_space=pltpu.VMEM)] * 5,
        out_specs=pl.BlockSpec(memory_space=pltpu.VMEM),
        scratch_shapes=[
            pltpu.VMEM((NSLOTS, 2, BH, DH, SKV), jnp.bfloat16),
            pltpu.SemaphoreType.DMA((N_DEV - 1,)),
            pltpu.SemaphoreType.DMA((N_DEV - 1,)),
        ],
        compiler_params=pltpu.CompilerParams(
            collective_id=0,
            vmem_limit_bytes=100 * 1024 * 1024,
        ),
    )(x, Wq, Wo, K_ext, V_ext)


# device time: 396316 ns/iter; 1.1903x vs baseline; 1.1903x over previous
import os

import jax
import jax.numpy as jnp
from jax import lax
from jax.experimental import pallas as pl
from jax.experimental.pallas import tpu as pltpu

_CACHE_DIR = os.path.join(os.path.dirname(os.path.abspath(__file__)), ".jaxcache")
os.makedirs(_CACHE_DIR, exist_ok=True)
jax.config.update("jax_compilation_cache_dir", _CACHE_DIR)
jax.config.update("jax_persistent_cache_min_compile_time_secs", 1.0)

N_DEV = 16
NSLOTS = 4
B, SQ, DM = 2, 256, 768
H, DH = 8, 64
SKV = 512
BH = B * H
SCALE = 0.125


def kernel(x, Wq, Wo, K_ext, V_ext):
    def body(x_ref, wq_ref, wo_ref, k_ref, v_ref, out_ref,
             kvbuf, send_sems, recv_sems):
        my = lax.axis_index("i")
        left = lax.rem(my + N_DEV - 1, N_DEV)
        right = lax.rem(my + 1, N_DEV)

        for b in range(B):
            for h in range(H):
                kvbuf[0, 0, b * H + h] = k_ref[b, :, h, :].astype(jnp.bfloat16).T
                kvbuf[0, 1, b * H + h] = v_ref[b, :, h, :].astype(jnp.bfloat16).T

        def make_hop(hop):
            s_slot = lax.rem(hop, NSLOTS)
            r_slot = lax.rem(hop + 1, NSLOTS)
            return pltpu.make_async_remote_copy(
                src_ref=kvbuf.at[s_slot],
                dst_ref=kvbuf.at[r_slot],
                send_sem=send_sems.at[hop],
                recv_sem=recv_sems.at[hop],
                device_id=(right,),
                device_id_type=pl.DeviceIdType.MESH,
            )

        barrier_sem = pltpu.get_barrier_semaphore()
        for nbr in (left, right):
            pl.semaphore_signal(
                barrier_sem, inc=1,
                device_id=(nbr,), device_id_type=pl.DeviceIdType.MESH,
            )
        pl.semaphore_wait(barrier_sem, 2)
        make_hop(0).start()

        xf = x_ref[...].reshape(B * SQ, DM).astype(jnp.bfloat16)
        q2d = lax.dot(
            xf, wq_ref[...].astype(jnp.bfloat16),
            preferred_element_type=jnp.float32,
        ).astype(jnp.bfloat16)
        q16 = jnp.concatenate(
            [q2d[b * SQ:(b + 1) * SQ, h * DH:(h + 1) * DH][None]
             for b in range(B) for h in range(H)],
            axis=0,
        )

        m = jnp.full((BH, SQ, 1), -jnp.inf, jnp.float32)
        l = jnp.zeros((BH, SQ, 1), jnp.float32)
        acc = jnp.zeros((BH, SQ, DH), jnp.float32)

        def chunk_update(slot, m, l, acc):
            kc = kvbuf[slot, 0]
            vc = kvbuf[slot, 1]
            s = lax.dot_general(
                q16, kc, (((2,), (1,)), ((0,), (0,))),
                preferred_element_type=jnp.float32,
            ) * SCALE
            mj = jnp.max(s, axis=-1, keepdims=True)
            m_new = jnp.maximum(m, mj)
            alpha = jnp.exp(m - m_new)
            p = jnp.exp(s - m_new)
            l_new = l * alpha + jnp.sum(p, axis=-1, keepdims=True)
            pv = lax.dot_general(
                p.astype(jnp.bfloat16), vc, (((2,), (2,)), ((0,), (0,))),
                preferred_element_type=jnp.float32,
            )
            return m_new, l_new, acc * alpha + pv

        m, l, acc = chunk_update(0, m, l, acc)

        def hop_body(hop, carry):
            m, l, acc = carry
            cur = make_hop(hop)
            cur.wait_recv()
            make_hop(hop + 1).start()
            cur.wait_send()
            return chunk_update(lax.rem(hop + 1, NSLOTS), m, l, acc)

        m, l, acc = lax.fori_loop(0, N_DEV - 2, hop_body, (m, l, acc))

        last = make_hop(N_DEV - 2)
        last.wait_recv()
        last.wait_send()
        m, l, acc = chunk_update((N_DEV - 1) % NSLOTS, m, l, acc)

        o = acc / l
        for b in range(B):
            ob = jnp.concatenate([o[b * H + h] for h in range(H)], axis=1)
            out_ref[b] = lax.dot(
                ob.astype(jnp.bfloat16), wo_ref[...].astype(jnp.bfloat16),
                preferred_element_type=jnp.float32,
            )

    return pl.pallas_call(
        body,
        out_shape=jax.ShapeDtypeStruct((B, SQ, DM), jnp.float32),
        in_specs=[pl.BlockSpec(memory_space=pltpu.VMEM)] * 5,
        out_specs=pl.BlockSpec(memory_space=pltpu.VMEM),
        scratch_shapes=[
            pltpu.VMEM((NSLOTS, 2, BH, DH, SKV), jnp.bfloat16),
            pltpu.SemaphoreType.DMA((N_DEV - 1,)),
            pltpu.SemaphoreType.DMA((N_DEV - 1,)),
        ],
        compiler_params=pltpu.CompilerParams(
            collective_id=0,
            vmem_limit_bytes=100 * 1024 * 1024,
        ),
    )(x, Wq, Wo, K_ext, V_ext)


# device time: 226974 ns/iter; 2.0783x vs baseline; 1.7461x over previous
import os

import jax
import jax.numpy as jnp
from jax import lax
from jax.experimental import pallas as pl
from jax.experimental.pallas import tpu as pltpu

_CACHE_DIR = os.path.join(os.path.dirname(os.path.abspath(__file__)), ".jaxcache")
os.makedirs(_CACHE_DIR, exist_ok=True)
jax.config.update("jax_compilation_cache_dir", _CACHE_DIR)
jax.config.update("jax_persistent_cache_min_compile_time_secs", 1.0)

N_DEV = 16
CW_HOPS = N_DEV // 2
CCW_HOPS = N_DEV - 1 - CW_HOPS
NSLOTS = 4
B, SQ, DM = 2, 256, 768
H, DH = 8, 64
SKV = 512
BH = B * H
SCALE = 0.125


def kernel(x, Wq, Wo, K_ext, V_ext):
    def body(x_ref, wq_ref, wo_ref, k_ref, v_ref, out_ref,
             kvbuf, cw_send, cw_recv, ccw_send, ccw_recv):
        my = lax.axis_index("i")
        left = lax.rem(my + N_DEV - 1, N_DEV)
        right = lax.rem(my + 1, N_DEV)

        for b in range(B):
            for h in range(H):
                kc = k_ref[b, :, h, :].astype(jnp.bfloat16).T
                vc = v_ref[b, :, h, :].astype(jnp.bfloat16).T
                kvbuf[0, 0, 0, b * H + h] = kc
                kvbuf[0, 0, 1, b * H + h] = vc
                kvbuf[1, 0, 0, b * H + h] = kc
                kvbuf[1, 0, 1, b * H + h] = vc

        def make_cw(hop):
            return pltpu.make_async_remote_copy(
                src_ref=kvbuf.at[0, lax.rem(hop, NSLOTS)],
                dst_ref=kvbuf.at[0, lax.rem(hop + 1, NSLOTS)],
                send_sem=cw_send.at[hop],
                recv_sem=cw_recv.at[hop],
                device_id=(right,),
                device_id_type=pl.DeviceIdType.MESH,
            )

        def make_ccw(hop):
            return pltpu.make_async_remote_copy(
                src_ref=kvbuf.at[1, lax.rem(hop, NSLOTS)],
                dst_ref=kvbuf.at[1, lax.rem(hop + 1, NSLOTS)],
                send_sem=ccw_send.at[hop],
                recv_sem=ccw_recv.at[hop],
                device_id=(left,),
                device_id_type=pl.DeviceIdType.MESH,
            )

        barrier_sem = pltpu.get_barrier_semaphore()
        for nbr in (left, right):
            pl.semaphore_signal(
                barrier_sem, inc=1,
                device_id=(nbr,), device_id_type=pl.DeviceIdType.MESH,
            )
        pl.semaphore_wait(barrier_sem, 2)
        make_cw(0).start()
        make_ccw(0).start()

        xf = x_ref[...].reshape(B * SQ, DM).astype(jnp.bfloat16)
        q2d = lax.dot(
            xf, wq_ref[...].astype(jnp.bfloat16),
            preferred_element_type=jnp.float32,
        ).astype(jnp.bfloat16)
        q16 = jnp.concatenate(
            [q2d[b * SQ:(b + 1) * SQ, h * DH:(h + 1) * DH][None]
             for b in range(B) for h in range(H)],
            axis=0,
        )

        m = jnp.full((BH, SQ, 1), -jnp.inf, jnp.float32)
        l = jnp.zeros((BH, SQ, 1), jnp.float32)
        acc = jnp.zeros((BH, SQ, DH), jnp.float32)

        def chunk_update(dirn, slot, m, l, acc):
            kc = kvbuf[dirn, slot, 0]
            vc = kvbuf[dirn, slot, 1]
            s = lax.dot_general(
                q16, kc, (((2,), (1,)), ((0,), (0,))),
                preferred_element_type=jnp.float32,
            ) * SCALE
            mj = jnp.max(s, axis=-1, keepdims=True)
            m_new = jnp.maximum(m, mj)
            alpha = jnp.exp(m - m_new)
            p = jnp.exp(s - m_new)
            l_new = l * alpha + jnp.sum(p, axis=-1, keepdims=True)
            pv = lax.dot_general(
                p.astype(jnp.bfloat16), vc, (((2,), (2,)), ((0,), (0,))),
                preferred_element_type=jnp.float32,
            )
            return m_new, l_new, acc * alpha + pv

        m, l, acc = chunk_update(0, 0, m, l, acc)

        def round_body(r, carry):
            m, l, acc = carry
            slot = lax.rem(r + 1, NSLOTS)
            cw = make_cw(r)
            cw.wait_recv()
            make_cw(r + 1).start()
            cw.wait_send()
            ccw = make_ccw(r)
            ccw.wait_recv()
            make_ccw(r + 1).start()
            ccw.wait_send()
            m, l, acc = chunk_update(0, slot, m, l, acc)
            return chunk_update(1, slot, m, l, acc)

        m, l, acc = lax.fori_loop(0, CCW_HOPS - 1, round_body, (m, l, acc))

        r6 = CCW_HOPS - 1
        slot6 = (r6 + 1) % NSLOTS
        cw = make_cw(r6)
        cw.wait_recv()
        make_cw(r6 + 1).start()
        cw.wait_send()
        ccw = make_ccw(r6)
        ccw.wait_recv()
        ccw.wait_send()
        m, l, acc = chunk_update(0, slot6, m, l, acc)
        m, l, acc = chunk_update(1, slot6, m, l, acc)

        r7 = CW_HOPS - 1
        cw = make_cw(r7)
        cw.wait_recv()
        cw.wait_send()
        m, l, acc = chunk_update(0, (r7 + 1) % NSLOTS, m, l, acc)

        o = acc / l
        for b in range(B):
            ob = jnp.concatenate([o[b * H + h] for h in range(H)], axis=1)
            out_ref[b] = lax.dot(
                ob.astype(jnp.bfloat16), wo_ref[...].astype(jnp.bfloat16),
                preferred_element_type=jnp.float32,
            )

    return pl.pallas_call(
        body,
        out_shape=jax.ShapeDtypeStruct((B, SQ, DM), jnp.float32),
        in_specs=[pl.BlockSpec(memory_space=pltpu.VMEM)] * 5,
        out_specs=pl.BlockSpec(memory_space=pltpu.VMEM),
        scratch_shapes=[
            pltpu.VMEM((2, NSLOTS, 2, BH, DH, SKV), jnp.bfloat16),
            pltpu.SemaphoreType.DMA((CW_HOPS,)),
            pltpu.SemaphoreType.DMA((CW_HOPS,)),
            pltpu.SemaphoreType.DMA((CCW_HOPS,)),
            pltpu.SemaphoreType.DMA((CCW_HOPS,)),
        ],
        compiler_params=pltpu.CompilerParams(
            collective_id=0,
            vmem_limit_bytes=100 * 1024 * 1024,
        ),
    )(x, Wq, Wo, K_ext, V_ext)


# device time: 213607 ns/iter; 2.2084x vs baseline; 1.0626x over previous
import os

import jax
import jax.numpy as jnp
from jax import lax
from jax.experimental import pallas as pl
from jax.experimental.pallas import tpu as pltpu

_CACHE_DIR = os.path.join(os.path.dirname(os.path.abspath(__file__)), ".jaxcache")
os.makedirs(_CACHE_DIR, exist_ok=True)
jax.config.update("jax_compilation_cache_dir", _CACHE_DIR)
jax.config.update("jax_persistent_cache_min_compile_time_secs", 1.0)

N_DEV = 16
CW_HOPS = N_DEV // 2
CCW_HOPS = N_DEV - 1 - CW_HOPS
NSLOTS = 4
B, SQ, DM = 2, 256, 768
H, DH = 8, 64
SKV = 512
BH = B * H
SCALE = 0.125


def kernel(x, Wq, Wo, K_ext, V_ext):
    def body(x_ref, wq_ref, wo_ref, k_ref, v_ref, out_ref,
             kvbuf, cw_send, cw_recv, ccw_send, ccw_recv):
        my = lax.axis_index("i")
        left = lax.rem(my + N_DEV - 1, N_DEV)
        right = lax.rem(my + 1, N_DEV)

        for b in range(B):
            for h in range(H):
                kc = k_ref[b, :, h, :].astype(jnp.bfloat16).T
                vc = v_ref[b, :, h, :].astype(jnp.bfloat16).T
                kvbuf[0, 0, 0, b * H + h] = kc
                kvbuf[0, 0, 1, b * H + h] = vc
                kvbuf[1, 0, 0, b * H + h] = kc
                kvbuf[1, 0, 1, b * H + h] = vc

        def make_cw(hop, kv):
            return pltpu.make_async_remote_copy(
                src_ref=kvbuf.at[0, lax.rem(hop, NSLOTS), kv],
                dst_ref=kvbuf.at[0, lax.rem(hop + 1, NSLOTS), kv],
                send_sem=cw_send.at[kv, hop],
                recv_sem=cw_recv.at[kv, hop],
                device_id=(right,),
                device_id_type=pl.DeviceIdType.MESH,
            )

        def make_ccw(hop, kv):
            return pltpu.make_async_remote_copy(
                src_ref=kvbuf.at[1, lax.rem(hop, NSLOTS), kv],
                dst_ref=kvbuf.at[1, lax.rem(hop + 1, NSLOTS), kv],
                send_sem=ccw_send.at[kv, hop],
                recv_sem=ccw_recv.at[kv, hop],
                device_id=(left,),
                device_id_type=pl.DeviceIdType.MESH,
            )

        barrier_sem = pltpu.get_barrier_semaphore()
        for nbr in (left, right):
            pl.semaphore_signal(
                barrier_sem, inc=1,
                device_id=(nbr,), device_id_type=pl.DeviceIdType.MESH,
            )
        pl.semaphore_wait(barrier_sem, 2)
        make_cw(0, 0).start()
        make_cw(0, 1).start()
        make_ccw(0, 0).start()
        make_ccw(0, 1).start()

        xf = x_ref[...].reshape(B * SQ, DM).astype(jnp.bfloat16)
        q2d = lax.dot(
            xf, wq_ref[...].astype(jnp.bfloat16),
            preferred_element_type=jnp.float32,
        ).astype(jnp.bfloat16)
        q16 = jnp.concatenate(
            [q2d[b * SQ:(b + 1) * SQ, h * DH:(h + 1) * DH][None]
             for b in range(B) for h in range(H)],
            axis=0,
        )

        m = jnp.full((BH, SQ, 1), -jnp.inf, jnp.float32)
        l = jnp.zeros((BH, SQ, 1), jnp.float32)
        acc = jnp.zeros((BH, SQ, DH), jnp.float32)

        def k_stage(dirn, slot, m, l):
            kc = kvbuf[dirn, slot, 0]
            s = lax.dot_general(
                q16, kc, (((2,), (1,)), ((0,), (0,))),
                preferred_element_type=jnp.float32,
            ) * SCALE
            mj = jnp.max(s, axis=-1, keepdims=True)
            m_new = jnp.maximum(m, mj)
            alpha = jnp.exp(m - m_new)
            p = jnp.exp(s - m_new)
            l_new = l * alpha + jnp.sum(p, axis=-1, keepdims=True)
            return m_new, l_new, alpha, p.astype(jnp.bfloat16)

        def v_stage(dirn, slot, alpha, p, acc):
            vc = kvbuf[dirn, slot, 1]
            pv = lax.dot_general(
                p, vc, (((2,), (2,)), ((0,), (0,))),
                preferred_element_type=jnp.float32,
            )
            return acc * alpha + pv

        def chunk_update(dirn, slot, m, l, acc):
            m, l, alpha, p = k_stage(dirn, slot, m, l)
            return m, l, v_stage(dirn, slot, alpha, p, acc)

        m, l, acc = chunk_update(0, 0, m, l, acc)

        def round_body(r, carry):
            m, l, acc = carry
            slot = lax.rem(r + 1, NSLOTS)
            cwk = make_cw(r, 0)
            cwk.wait_recv()
            make_cw(r + 1, 0).start()
            cwk.wait_send()
            ccwk = make_ccw(r, 0)
            ccwk.wait_recv()
            make_ccw(r + 1, 0).start()
            ccwk.wait_send()
            m, l, a_cw, p_cw = k_stage(0, slot, m, l)
            cwv = make_cw(r, 1)
            cwv.wait_recv()
            make_cw(r + 1, 1).start()
            cwv.wait_send()
            acc = v_stage(0, slot, a_cw, p_cw, acc)
            m, l, a_ccw, p_ccw = k_stage(1, slot, m, l)
            ccwv = make_ccw(r, 1)
            ccwv.wait_recv()
            make_ccw(r + 1, 1).start()
            ccwv.wait_send()
            acc = v_stage(1, slot, a_ccw, p_ccw, acc)
            return m, l, acc

        m, l, acc = lax.fori_loop(0, CCW_HOPS - 1, round_body, (m, l, acc))

        r6 = CCW_HOPS - 1
        slot6 = (r6 + 1) % NSLOTS
        cwk = make_cw(r6, 0)
        cwk.wait_recv()
        make_cw(r6 + 1, 0).start()
        cwk.wait_send()
        ccwk = make_ccw(r6, 0)
        ccwk.wait_recv()
        ccwk.wait_send()
        m, l, a_cw, p_cw = k_stage(0, slot6, m, l)
        cwv = make_cw(r6, 1)
        cwv.wait_recv()
        make_cw(r6 + 1, 1).start()
        cwv.wait_send()
        acc = v_stage(0, slot6, a_cw, p_cw, acc)
        m, l, a_ccw, p_ccw = k_stage(1, slot6, m, l)
        ccwv = make_ccw(r6, 1)
        ccwv.wait_recv()
        ccwv.wait_send()
        acc = v_stage(1, slot6, a_ccw, p_ccw, acc)

        r7 = CW_HOPS - 1
        slot7 = (r7 + 1) % NSLOTS
        cwk = make_cw(r7, 0)
        cwk.wait_recv()
        cwk.wait_send()
        m, l, a_cw, p_cw = k_stage(0, slot7, m, l)
        cwv = make_cw(r7, 1)
        cwv.wait_recv()
        cwv.wait_send()
        acc = v_stage(0, slot7, a_cw, p_cw, acc)

        o = acc / l
        for b in range(B):
            ob = jnp.concatenate([o[b * H + h] for h in range(H)], axis=1)
            out_ref[b] = lax.dot(
                ob.astype(jnp.bfloat16), wo_ref[...].astype(jnp.bfloat16),
                preferred_element_type=jnp.float32,
            )

    return pl.pallas_call(
        body,
        out_shape=jax.ShapeDtypeStruct((B, SQ, DM), jnp.float32),
        in_specs=[pl.BlockSpec(memory_space=pltpu.VMEM)] * 5,
        out_specs=pl.BlockSpec(memory_space=pltpu.VMEM),
        scratch_shapes=[
            pltpu.VMEM((2, NSLOTS, 2, BH, DH, SKV), jnp.bfloat16),
            pltpu.SemaphoreType.DMA((2, CW_HOPS)),
            pltpu.SemaphoreType.DMA((2, CW_HOPS)),
            pltpu.SemaphoreType.DMA((2, CCW_HOPS)),
            pltpu.SemaphoreType.DMA((2, CCW_HOPS)),
        ],
        compiler_params=pltpu.CompilerParams(
            collective_id=0,
            vmem_limit_bytes=100 * 1024 * 1024,
        ),
    )(x, Wq, Wo, K_ext, V_ext)


# device time: 208301 ns/iter; 2.2646x vs baseline; 1.0255x over previous
import os

import jax
import jax.numpy as jnp
from jax import lax
from jax.experimental import pallas as pl
from jax.experimental.pallas import tpu as pltpu

_CACHE_DIR = os.path.join(os.path.dirname(os.path.abspath(__file__)), ".jaxcache")
os.makedirs(_CACHE_DIR, exist_ok=True)
jax.config.update("jax_compilation_cache_dir", _CACHE_DIR)
jax.config.update("jax_persistent_cache_min_compile_time_secs", 1.0)

N_DEV = 16
CW_HOPS = N_DEV // 2
CCW_HOPS = N_DEV - 1 - CW_HOPS
NSLOTS = 4
B, SQ, DM = 2, 256, 768
H, DH = 8, 64
SKV = 512
BH = B * H
SCALE = 0.125


def kernel(x, Wq, Wo, K_ext, V_ext):
    def body(x_ref, wq_ref, wo_ref, k_ref, v_ref, out_ref,
             kvbuf, cw_send, cw_recv, ccw_send, ccw_recv):
        my = lax.axis_index("i")
        left = lax.rem(my + N_DEV - 1, N_DEV)
        right = lax.rem(my + 1, N_DEV)

        for b in range(B):
            for h in range(H):
                kc = k_ref[b, :, h, :].astype(jnp.bfloat16).T
                kvbuf[0, 0, 0, b * H + h] = kc
                kvbuf[1, 0, 0, b * H + h] = kc

        def make_cw(hop, kv):
            return pltpu.make_async_remote_copy(
                src_ref=kvbuf.at[0, lax.rem(hop, NSLOTS), kv],
                dst_ref=kvbuf.at[0, lax.rem(hop + 1, NSLOTS), kv],
                send_sem=cw_send.at[kv, hop],
                recv_sem=cw_recv.at[kv, hop],
                device_id=(right,),
                device_id_type=pl.DeviceIdType.MESH,
            )

        def make_ccw(hop, kv):
            return pltpu.make_async_remote_copy(
                src_ref=kvbuf.at[1, lax.rem(hop, NSLOTS), kv],
                dst_ref=kvbuf.at[1, lax.rem(hop + 1, NSLOTS), kv],
                send_sem=ccw_send.at[kv, hop],
                recv_sem=ccw_recv.at[kv, hop],
                device_id=(left,),
                device_id_type=pl.DeviceIdType.MESH,
            )

        barrier_sem = pltpu.get_barrier_semaphore()
        for nbr in (left, right):
            pl.semaphore_signal(
                barrier_sem, inc=1,
                device_id=(nbr,), device_id_type=pl.DeviceIdType.MESH,
            )
        pl.semaphore_wait(barrier_sem, 2)
        make_cw(0, 0).start()
        make_ccw(0, 0).start()

        for b in range(B):
            for h in range(H):
                vc = v_ref[b, :, h, :].astype(jnp.bfloat16).T
                kvbuf[0, 0, 1, b * H + h] = vc
                kvbuf[1, 0, 1, b * H + h] = vc
        make_cw(0, 1).start()
        make_ccw(0, 1).start()

        xf = x_ref[...].reshape(B * SQ, DM).astype(jnp.bfloat16)
        q2d = lax.dot(
            xf, wq_ref[...].astype(jnp.bfloat16),
            preferred_element_type=jnp.float32,
        ).astype(jnp.bfloat16)
        q16 = jnp.concatenate(
            [q2d[b * SQ:(b + 1) * SQ, h * DH:(h + 1) * DH][None]
             for b in range(B) for h in range(H)],
            axis=0,
        )

        m = jnp.full((BH, SQ, 1), -jnp.inf, jnp.float32)
        l = jnp.zeros((BH, SQ, 1), jnp.float32)
        acc = jnp.zeros((BH, SQ, DH), jnp.float32)

        def k_stage(dirn, slot, m, l):
            kc = kvbuf[dirn, slot, 0]
            s = lax.dot_general(
                q16, kc, (((2,), (1,)), ((0,), (0,))),
                preferred_element_type=jnp.float32,
            ) * SCALE
            mj = jnp.max(s, axis=-1, keepdims=True)
            m_new = jnp.maximum(m, mj)
            alpha = jnp.exp(m - m_new)
            p = jnp.exp(s - m_new)
            l_new = l * alpha + jnp.sum(p, axis=-1, keepdims=True)
            return m_new, l_new, alpha, p.astype(jnp.bfloat16)

        def v_stage(dirn, slot, alpha, p, acc):
            vc = kvbuf[dirn, slot, 1]
            pv = lax.dot_general(
                p, vc, (((2,), (2,)), ((0,), (0,))),
                preferred_element_type=jnp.float32,
            )
            return acc * alpha + pv

        def chunk_update(dirn, slot, m, l, acc):
            m, l, alpha, p = k_stage(dirn, slot, m, l)
            return m, l, v_stage(dirn, slot, alpha, p, acc)

        m, l, acc = chunk_update(0, 0, m, l, acc)

        def round_body(r, carry):
            m, l, acc = carry
            slot = lax.rem(r + 1, NSLOTS)
            cwk = make_cw(r, 0)
            cwk.wait_recv()
            make_cw(r + 1, 0).start()
            cwk.wait_send()
            ccwk = make_ccw(r, 0)
            ccwk.wait_recv()
            make_ccw(r + 1, 0).start()
            ccwk.wait_send()
            m, l, a_cw, p_cw = k_stage(0, slot, m, l)
            cwv = make_cw(r, 1)
            cwv.wait_recv()
            make_cw(r + 1, 1).start()
            cwv.wait_send()
            acc = v_stage(0, slot, a_cw, p_cw, acc)
            m, l, a_ccw, p_ccw = k_stage(1, slot, m, l)
            ccwv = make_ccw(r, 1)
            ccwv.wait_recv()
            make_ccw(r + 1, 1).start()
            ccwv.wait_send()
            acc = v_stage(1, slot, a_ccw, p_ccw, acc)
            return m, l, acc

        m, l, acc = lax.fori_loop(0, CCW_HOPS - 1, round_body, (m, l, acc))

        r6 = CCW_HOPS - 1
        slot6 = (r6 + 1) % NSLOTS
        cwk = make_cw(r6, 0)
        cwk.wait_recv()
        make_cw(r6 + 1, 0).start()
        cwk.wait_send()
        ccwk = make_ccw(r6, 0)
        ccwk.wait_recv()
        ccwk.wait_send()
        m, l, a_cw, p_cw = k_stage(0, slot6, m, l)
        cwv = make_cw(r6, 1)
        cwv.wait_recv()
        make_cw(r6 + 1, 1).start()
        cwv.wait_send()
        acc = v_stage(0, slot6, a_cw, p_cw, acc)
        m, l, a_ccw, p_ccw = k_stage(1, slot6, m, l)
        ccwv = make_ccw(r6, 1)
        ccwv.wait_recv()
        ccwv.wait_send()
        acc = v_stage(1, slot6, a_ccw, p_ccw, acc)

        r7 = CW_HOPS - 1
        slot7 = (r7 + 1) % NSLOTS
        cwk = make_cw(r7, 0)
        cwk.wait_recv()
        cwk.wait_send()
        m, l, a_cw, p_cw = k_stage(0, slot7, m, l)
        cwv = make_cw(r7, 1)
        cwv.wait_recv()
        cwv.wait_send()
        acc = v_stage(0, slot7, a_cw, p_cw, acc)

        o = acc / l
        for b in range(B):
            ob = jnp.concatenate([o[b * H + h] for h in range(H)], axis=1)
            out_ref[b] = lax.dot(
                ob.astype(jnp.bfloat16), wo_ref[...].astype(jnp.bfloat16),
                preferred_element_type=jnp.float32,
            )

    return pl.pallas_call(
        body,
        out_shape=jax.ShapeDtypeStruct((B, SQ, DM), jnp.float32),
        in_specs=[pl.BlockSpec(memory_space=pltpu.VMEM)] * 5,
        out_specs=pl.BlockSpec(memory_space=pltpu.VMEM),
        scratch_shapes=[
            pltpu.VMEM((2, NSLOTS, 2, BH, DH, SKV), jnp.bfloat16),
            pltpu.SemaphoreType.DMA((2, CW_HOPS)),
            pltpu.SemaphoreType.DMA((2, CW_HOPS)),
            pltpu.SemaphoreType.DMA((2, CCW_HOPS)),
            pltpu.SemaphoreType.DMA((2, CCW_HOPS)),
        ],
        compiler_params=pltpu.CompilerParams(
            collective_id=0,
            vmem_limit_bytes=100 * 1024 * 1024,
        ),
    )(x, Wq, Wo, K_ext, V_ext)
